# baseline (speedup 1.0000x reference)
import math

import jax
import jax.numpy as jnp
import numpy as np
from jax import lax
from jax.experimental import pallas as pl
from jax.experimental.pallas import tpu as pltpu

_Q_FRACTION = 0.15
_NPROBE = 128
_NROUNDS = 5
_NCHUNK = 8


def _num_top_table(t: int) -> np.ndarray:
    return np.array(
        [max(1, int(math.ceil(_Q_FRACTION * n))) for n in range(t + 1)],
        dtype=np.int32,
    )


def _pool_body(len_ref, table_ref, x_ref, o_ref, xkeep_ref, sqkeep_ref):
    b = pl.program_id(0)
    c = pl.program_id(1)
    CR = x_ref.shape[1]
    D = x_ref.shape[2]
    T = CR * _NCHUNK
    R = T // 128
    RC = CR // 128

    @pl.when(c < _NCHUNK)
    def _stream_chunk():
        xc = x_ref[0]
        xkeep_ref[pl.ds(c * CR, CR), :] = xc
        xc3 = xc.reshape(RC, 128, D)
        sqkeep_ref[c] = jnp.sum(xc3 * xc3, axis=-1)

    @pl.when(c == _NCHUNK)
    def _finalize():
        L = len_ref[b]
        nt = table_ref[L]

        sq = sqkeep_ref[...].reshape(R, 128)
        nrm = jnp.sqrt(sq)

        ri = lax.broadcasted_iota(jnp.int32, (R, 128), 0)
        ci = lax.broadcasted_iota(jnp.int32, (R, 128), 1)
        t_idx = ri * 128 + ci
        valid = t_idx < L

        key = jnp.where(valid, lax.bitcast_convert_type(nrm, jnp.int32), -1)

        pidx = lax.broadcasted_iota(jnp.int32, (1, _NPROBE), 1)
        big = jnp.int32(2**31 - 1)

        def round_fn(_, carry):
            lo, hi = carry
            w = hi - lo - 1
            s = jnp.maximum((w + _NPROBE - 1) // _NPROBE, 1)
            probes = jnp.minimum(lo + 1 + pidx * s, hi - 1)
            ge = (key[:, :, None] >= probes[None, :, :]).astype(jnp.int32)
            cnts = jnp.sum(ge, axis=(0, 1))[None, :]
            ok = cnts >= nt
            new_lo = jnp.maximum(lo, jnp.max(jnp.where(ok, probes, -1),
                                             axis=1, keepdims=True))
            new_hi = jnp.minimum(hi, jnp.min(jnp.where(ok, big, probes),
                                             axis=1, keepdims=True))
            return new_lo, new_hi

        lo0 = jnp.full((1, 1), -1, jnp.int32)
        hi0 = jnp.full((1, 1), 0x7F800001, jnp.int32)
        th, _ = lax.fori_loop(0, _NROUNDS, round_fn, (lo0, hi0))

        gt = key > th
        eq = key == th
        c_gt = jnp.sum(gt.astype(jnp.int32))
        r = nt - c_gt

        eqf = eq.astype(jnp.float32)
        cj = lax.broadcasted_iota(jnp.int32, (128, 128), 0)
        ck = lax.broadcasted_iota(jnp.int32, (128, 128), 1)
        strict_ut = (cj < ck).astype(jnp.float32)
        inrow_exc = lax.dot_general(eqf, strict_ut, (((1,), (0,)), ((), ())))
        rows = jnp.sum(eqf, axis=1, keepdims=True)
        rj = lax.broadcasted_iota(jnp.int32, (R, R), 0)
        rk = lax.broadcasted_iota(jnp.int32, (R, R), 1)
        strict_lt = (rk < rj).astype(jnp.float32)
        rows_exc = lax.dot_general(strict_lt, rows, (((1,), (0,)), ((), ())))
        exc = inrow_exc + rows_exc

        sel = gt | (eq & (exc < r.astype(jnp.float32)))
        w_row = sel.astype(jnp.float32).reshape(1, T)

        pooled = lax.dot_general(w_row, xkeep_ref[...],
                                 (((1,), (0,)), ((), ())),
                                 precision=lax.Precision.HIGHEST)
        o_ref[0, 0, :] = pooled[0] / nt.astype(jnp.float32)


def kernel(x, lengths):
    B, T, D = x.shape
    CR = T // _NCHUNK
    table = jnp.asarray(_num_top_table(T))
    return pl.pallas_call(
        _pool_body,
        grid=(B, _NCHUNK + 1),
        in_specs=[
            pl.BlockSpec(memory_space=pltpu.SMEM),
            pl.BlockSpec(memory_space=pltpu.SMEM),
            pl.BlockSpec((1, CR, D),
                         lambda b, c: (b, jnp.minimum(c, _NCHUNK - 1), 0)),
        ],
        out_specs=pl.BlockSpec((1, 1, D), lambda b, c: (b, 0, 0)),
        out_shape=jax.ShapeDtypeStruct((B, 1, D), jnp.float32),
        scratch_shapes=[
            pltpu.VMEM((T, D), jnp.float32),
            pltpu.VMEM((_NCHUNK, T // _NCHUNK // 128, 128), jnp.float32),
        ],
    )(lengths, table, x).reshape(B, D)
